# Initial kernel scaffold; baseline (speedup 1.0000x reference)
import jax
import jax.numpy as jnp
from jax.experimental import pallas as pl


def kernel(x, weight, expert_bias):
    raise NotImplementedError("write your pallas kernel here")


# speedup vs baseline: 2.9577x; 2.9577x over previous
import functools

import jax
import jax.numpy as jnp
from jax import lax
from jax.experimental import pallas as pl
from jax.experimental.pallas import tpu as pltpu
from jax.experimental.pallas import tpu_sc as plsc

N = 32768
D = 768
E = 8
ROUTE_SCALE = 2.5

NC = 2
NS = 16
L = 16
NW = NC * NS
TPW = N // NW
TILE = 1024


def _mm_body(x_ref, wt_ref, s_ref):
    z = jnp.dot(x_ref[...], wt_ref[...], preferred_element_type=jnp.float32)
    s_ref[...] = 1.0 / (1.0 + jnp.exp(-z))


def _scores_tc(x, wt):
    return pl.pallas_call(
        _mm_body,
        grid=(N // TILE,),
        in_specs=[
            pl.BlockSpec((TILE, D), lambda i: (i, 0)),
            pl.BlockSpec((D, E), lambda i: (0, 0)),
        ],
        out_specs=pl.BlockSpec((TILE, E), lambda i: (i, 0)),
        out_shape=jax.ShapeDtypeStruct((N, E), jnp.float32),
    )(x, wt)


@functools.partial(
    pl.kernel,
    out_type=(
        jax.ShapeDtypeStruct((N * 2,), jnp.float32),
        jax.ShapeDtypeStruct((N * 2,), jnp.int32),
    ),
    mesh=plsc.VectorSubcoreMesh(core_axis_name="c", subcore_axis_name="s"),
    compiler_params=pltpu.CompilerParams(needs_layout_passes=False),
    scratch_types=[
        pltpu.VMEM((TPW * E,), jnp.float32),
        pltpu.VMEM((L,), jnp.float32),
        pltpu.VMEM((TPW * 2,), jnp.float32),
        pltpu.VMEM((TPW * 2,), jnp.int32),
    ],
)
def _route_sc(scores_hbm, bias_hbm, w_hbm, i_hbm, s_v, b_v, w_v, i_v):
    wid = lax.axis_index("s") * NC + lax.axis_index("c")
    base = wid * TPW
    pltpu.sync_copy(scores_hbm.at[pl.ds(base * E, TPW * E)], s_v)
    pltpu.sync_copy(bias_hbm, b_v)

    iota = lax.iota(jnp.int32, L)
    bias = [plsc.load_gather(b_v, [jnp.full((L,), e, jnp.int32)]) for e in range(E)]
    NEG = jnp.float32(-jnp.inf)

    def top2sum(a0, a1, a2, a3):
        return jnp.maximum(
            jnp.maximum(jnp.maximum(a0, a1) + jnp.maximum(a2, a3), a0 + a1),
            a2 + a3,
        )

    def body(i, carry):
        row = i * L + iota
        row8 = row * E
        s = [plsc.load_gather(s_v, [row8 + e]) for e in range(E)]
        b = [s[e] + bias[e] for e in range(E)]
        g0 = top2sum(b[0], b[1], b[2], b[3])
        g1 = top2sum(b[4], b[5], b[6], b[7])
        use1 = g1 > g0
        c = [jnp.where(use1, b[4 + j], b[j]) for j in range(4)]
        o = [jnp.where(use1, s[4 + j], s[j]) for j in range(4)]
        m1 = jnp.maximum(jnp.maximum(c[0], c[1]), jnp.maximum(c[2], c[3]))
        e0, e1, e2 = c[0] == m1, c[1] == m1, c[2] == m1
        idx1 = jnp.where(e0, 0, jnp.where(e1, 1, jnp.where(e2, 2, 3)))
        idx1 = idx1.astype(jnp.int32)
        cp = [jnp.where(idx1 == j, NEG, c[j]) for j in range(4)]
        m2 = jnp.maximum(jnp.maximum(cp[0], cp[1]), jnp.maximum(cp[2], cp[3]))
        f0, f1, f2 = cp[0] == m2, cp[1] == m2, cp[2] == m2
        idx2 = jnp.where(f0, 0, jnp.where(f1, 1, jnp.where(f2, 2, 3)))
        idx2 = idx2.astype(jnp.int32)
        w1 = jnp.where(e0, o[0], jnp.where(e1, o[1], jnp.where(e2, o[2], o[3])))
        w2 = jnp.where(f0, o[0], jnp.where(f1, o[1], jnp.where(f2, o[2], o[3])))
        inv = jnp.float32(ROUTE_SCALE) / (w1 + w2 + jnp.float32(1e-20))
        gbase = jnp.where(use1, 4, 0).astype(jnp.int32)
        row2 = row * 2
        plsc.store_scatter(w_v, [row2], w1 * inv)
        plsc.store_scatter(w_v, [row2 + 1], w2 * inv)
        plsc.store_scatter(i_v, [row2], gbase + idx1)
        plsc.store_scatter(i_v, [row2 + 1], gbase + idx2)
        return carry

    lax.fori_loop(0, TPW // L, body, 0)
    pltpu.sync_copy(w_v, w_hbm.at[pl.ds(base * 2, TPW * 2)])
    pltpu.sync_copy(i_v, i_hbm.at[pl.ds(base * 2, TPW * 2)])


def kernel(x, weight, expert_bias):
    wt = weight.T
    scores = _scores_tc(x, wt)
    bias16 = jnp.zeros((L,), jnp.float32).at[:E].set(expert_bias)
    w, i = _route_sc(scores.reshape(-1), bias16)
    return w.reshape(N, 2).astype(x.dtype), i.reshape(N, 2)


# speedup vs baseline: 3.1862x; 1.0772x over previous
import functools

import jax
import jax.numpy as jnp
from jax import lax
from jax.experimental import pallas as pl
from jax.experimental.pallas import tpu as pltpu
from jax.experimental.pallas import tpu_sc as plsc

N = 32768
D = 768
E = 8
ROUTE_SCALE = 2.5

NC = 2
NS = 16
L = 16
NW = NC * NS
TPW = N // NW
TILE = 4096


def _mm_body(x_ref, wt_ref, s_ref):
    z = jnp.dot(x_ref[...], wt_ref[...], preferred_element_type=jnp.float32)
    s_ref[...] = 1.0 / (1.0 + jnp.exp(-z))


def _scores_tc(x, wt):
    return pl.pallas_call(
        _mm_body,
        grid=(N // TILE,),
        in_specs=[
            pl.BlockSpec((TILE, D), lambda i: (i, 0)),
            pl.BlockSpec((D, E), lambda i: (0, 0)),
        ],
        out_specs=pl.BlockSpec((TILE, E), lambda i: (i, 0)),
        out_shape=jax.ShapeDtypeStruct((N, E), jnp.float32),
    )(x, wt)


@functools.partial(
    pl.kernel,
    out_type=(
        jax.ShapeDtypeStruct((N * 2,), jnp.float32),
        jax.ShapeDtypeStruct((N * 2,), jnp.int32),
    ),
    mesh=plsc.VectorSubcoreMesh(core_axis_name="c", subcore_axis_name="s"),
    compiler_params=pltpu.CompilerParams(needs_layout_passes=False),
    scratch_types=[
        pltpu.VMEM((TPW * E,), jnp.float32),
        pltpu.VMEM((L,), jnp.float32),
        pltpu.VMEM((TPW * 2,), jnp.float32),
        pltpu.VMEM((TPW * 2,), jnp.int32),
    ],
)
def _route_sc(scores_hbm, bias_hbm, w_hbm, i_hbm, s_v, b_v, w_v, i_v):
    wid = lax.axis_index("s") * NC + lax.axis_index("c")
    base = wid * TPW
    pltpu.sync_copy(scores_hbm.at[pl.ds(base * E, TPW * E)], s_v)
    pltpu.sync_copy(bias_hbm, b_v)

    iota = lax.iota(jnp.int32, L)
    bias = [plsc.load_gather(b_v, [jnp.full((L,), e, jnp.int32)]) for e in range(E)]
    NEG = jnp.float32(-jnp.inf)

    def top2sum(a0, a1, a2, a3):
        return jnp.maximum(
            jnp.maximum(jnp.maximum(a0, a1) + jnp.maximum(a2, a3), a0 + a1),
            a2 + a3,
        )

    def body(i, carry):
        row = i * L + iota
        row8 = row * E
        s = [plsc.load_gather(s_v, [row8 + e]) for e in range(E)]
        b = [s[e] + bias[e] for e in range(E)]
        g0 = top2sum(b[0], b[1], b[2], b[3])
        g1 = top2sum(b[4], b[5], b[6], b[7])
        use1 = g1 > g0
        c = [jnp.where(use1, b[4 + j], b[j]) for j in range(4)]
        o = [jnp.where(use1, s[4 + j], s[j]) for j in range(4)]
        m1 = jnp.maximum(jnp.maximum(c[0], c[1]), jnp.maximum(c[2], c[3]))
        e0, e1, e2 = c[0] == m1, c[1] == m1, c[2] == m1
        idx1 = jnp.where(e0, 0, jnp.where(e1, 1, jnp.where(e2, 2, 3)))
        idx1 = idx1.astype(jnp.int32)
        cp = [jnp.where(idx1 == j, NEG, c[j]) for j in range(4)]
        m2 = jnp.maximum(jnp.maximum(cp[0], cp[1]), jnp.maximum(cp[2], cp[3]))
        f0, f1, f2 = cp[0] == m2, cp[1] == m2, cp[2] == m2
        idx2 = jnp.where(f0, 0, jnp.where(f1, 1, jnp.where(f2, 2, 3)))
        idx2 = idx2.astype(jnp.int32)
        w1 = jnp.where(e0, o[0], jnp.where(e1, o[1], jnp.where(e2, o[2], o[3])))
        w2 = jnp.where(f0, o[0], jnp.where(f1, o[1], jnp.where(f2, o[2], o[3])))
        inv = jnp.float32(ROUTE_SCALE) / (w1 + w2 + jnp.float32(1e-20))
        gbase = jnp.where(use1, 4, 0).astype(jnp.int32)
        row2 = row * 2
        plsc.store_scatter(w_v, [row2], w1 * inv)
        plsc.store_scatter(w_v, [row2 + 1], w2 * inv)
        plsc.store_scatter(i_v, [row2], gbase + idx1)
        plsc.store_scatter(i_v, [row2 + 1], gbase + idx2)
        return carry

    lax.fori_loop(0, TPW // L, body, 0)
    pltpu.sync_copy(w_v, w_hbm.at[pl.ds(base * 2, TPW * 2)])
    pltpu.sync_copy(i_v, i_hbm.at[pl.ds(base * 2, TPW * 2)])


def kernel(x, weight, expert_bias):
    wt = weight.T
    scores = _scores_tc(x, wt)
    bias16 = jnp.zeros((L,), jnp.float32).at[:E].set(expert_bias)
    w, i = _route_sc(scores.reshape(-1), bias16)
    return w.reshape(N, 2).astype(x.dtype), i.reshape(N, 2)
